# baseline (speedup 1.0000x reference)
import functools

import jax
import jax.numpy as jnp
from jax import lax
from jax.experimental import pallas as pl
from jax.experimental.pallas import tpu as pltpu
from jax.experimental.pallas import tpu_sc as plsc

B = 4096
D = 32
NC = 2
NS = 16
NW = NC * NS
BPW = B // NW


def _sc_gather_body(users_hbm, items_hbm, uf_hbm, if_hbm, u_out, v_out,
                    uidx, urows, iidx, irows, usem, isem):
    wid = lax.axis_index("s") * NC + lax.axis_index("c")
    base = wid * BPW
    pltpu.sync_copy(users_hbm.at[pl.ds(base, BPW)], uidx)
    pltpu.sync_copy(items_hbm.at[pl.ds(base, BPW)], iidx)
    cu = pltpu.async_copy(uf_hbm.at[uidx], urows, usem)
    ci = pltpu.async_copy(if_hbm.at[iidx], irows, isem)
    cu.wait()
    ci.wait()
    pltpu.sync_copy(urows, u_out.at[pl.ds(base, BPW)])
    pltpu.sync_copy(irows, v_out.at[pl.ds(base, BPW)])


_sc_gather = functools.partial(
    pl.kernel,
    mesh=plsc.VectorSubcoreMesh(core_axis_name="c", subcore_axis_name="s"),
    out_type=[
        jax.ShapeDtypeStruct((B, D), jnp.float32),
        jax.ShapeDtypeStruct((B, D), jnp.float32),
    ],
    scratch_types=[
        pltpu.VMEM((BPW,), jnp.int32),
        pltpu.VMEM((BPW, D), jnp.float32),
        pltpu.VMEM((BPW,), jnp.int32),
        pltpu.VMEM((BPW, D), jnp.float32),
        pltpu.SemaphoreType.DMA,
        pltpu.SemaphoreType.DMA,
    ],
    compiler_params=pltpu.CompilerParams(use_tc_tiling_on_sc=False),
)(_sc_gather_body)


def _mm_body(u_ref, v_ref, o_ref):
    o_ref[...] = lax.dot_general(
        u_ref[...], v_ref[...],
        (((1,), (1,)), ((), ())),
        preferred_element_type=jnp.float32,
    )


BM = 512
BN = 1024


def _tc_matmul(u, v):
    return pl.pallas_call(
        _mm_body,
        grid=(B // BM, B // BN),
        in_specs=[
            pl.BlockSpec((BM, D), lambda i, j: (i, 0)),
            pl.BlockSpec((BN, D), lambda i, j: (j, 0)),
        ],
        out_specs=pl.BlockSpec((BM, BN), lambda i, j: (i, j)),
        out_shape=jax.ShapeDtypeStruct((B, B), jnp.float32),
    )(u, v)


def kernel(users, items, user_factors, item_factors):
    u, v = _sc_gather(users.astype(jnp.int32), items.astype(jnp.int32),
                      user_factors, item_factors)
    return _tc_matmul(u, v)
